# baseline (speedup 1.0000x reference)
import functools

import numpy as np

import jax
import jax.numpy as jnp
from jax import lax
from jax.experimental import pallas as pl
from jax.experimental.pallas import tpu as pltpu
from jax.experimental.pallas import tpu_sc as plsc

N = 10000
F = 128
E = 320000
NC = 2
NS = 16
NW = NC * NS
CH = 128
NCHUNKS = -(-E // CH)
KCH0 = int(round(NCHUNKS * 0.38 / NS))
KCH1 = -(-(NCHUNKS - KCH0 * NS) // NS)
KMAX = max(KCH0, KCH1)
SHALF = (-(-KMAX // 2) + 7) // 8 * 8
KPAD = 2 * SHALF
ROWS_PER_TILE = (N // NS) // 8 * 8
TAIL_ROWS = N - NS * ROWS_PER_TILE
TAIL_BASE = NS * ROWS_PER_TILE


def _prep_body(x_ref, degc_ref, w_ref, c_ref,
               wxi_ref, whi_ref, bi_ref, wxf_ref, whf_ref, bf_ref,
               wxg_ref, whg_ref, bg_ref, wxo_ref, who_ref, bo_ref,
               table_ref):
    W = w_ref[...]
    dot = functools.partial(jnp.dot, preferred_element_type=jnp.float32,
                            precision=lax.Precision.HIGHEST)
    i_g = jax.nn.sigmoid(dot(W, wxi_ref[...] + whi_ref[...]) + bi_ref[...])
    f_g = jax.nn.sigmoid(dot(W, wxf_ref[...] + whf_ref[...]) + bf_ref[...])
    g_g = jnp.tanh(dot(W, wxg_ref[...] + whg_ref[...]) + bg_ref[...])
    o_g = jax.nn.sigmoid(dot(W, wxo_ref[...] + who_ref[...]) + bo_ref[...])
    c_new = f_g * c_ref[...] + i_g * g_g
    W_new = o_g * jnp.tanh(c_new)
    dinv = lax.rsqrt(jnp.maximum(degc_ref[...], 1e-6))
    table_ref[...] = dot(x_ref[...] * dinv, W_new)


_prep = pl.pallas_call(
    _prep_body,
    out_shape=jax.ShapeDtypeStruct((N, F), jnp.float32),
)


@functools.cache
def _make_sc_scatter():
    mesh = plsc.VectorSubcoreMesh(core_axis_name="c", subcore_axis_name="s")
    return pl.kernel(
        _sc_scatter_body,
        out_type=jax.ShapeDtypeStruct((NC, N, F), jnp.float32),
        mesh=mesh,
        scratch_types=[
            pltpu.VMEM((SHALF, CH), jnp.int32),
            pltpu.VMEM((SHALF, CH), jnp.int32),
            pltpu.VMEM((SHALF, CH), jnp.float32),
            pltpu.VMEM((CH, F), jnp.float32),
            pltpu.VMEM_SHARED((N, F), jnp.float32),
            pltpu.SemaphoreType.DMA,
        ],
    )


def _sc_scatter_body(table_hbm, srcp_hbm, dstp_hbm, ewp_hbm, out_hbm,
                     src_v, dst_v, ew_v, rows_v, acc, sem):
    c = lax.axis_index("c")
    s = lax.axis_index("s")
    wid = s * NC + c
    pltpu.sync_copy(srcp_hbm.at[wid].at[pl.ds(0, SHALF)], src_v)
    pltpu.sync_copy(dstp_hbm.at[wid].at[pl.ds(0, SHALF)], dst_v)
    pltpu.sync_copy(ewp_hbm.at[wid].at[pl.ds(0, SHALF)], ew_v)

    zvec = jnp.zeros((16,), jnp.float32)

    def _zrow(i, carry):
        for j in range(8):
            rows_v[i, pl.ds(j * 16, 16)] = zvec
        return carry

    lax.fori_loop(0, CH, _zrow, 0)
    base = s * ROWS_PER_TILE
    full, rem = divmod(ROWS_PER_TILE, CH)
    for t in range(full):
        pltpu.sync_copy(rows_v, acc.at[pl.ds(base + t * CH, CH)])
    if rem:
        pltpu.sync_copy(rows_v.at[pl.ds(0, rem)],
                        acc.at[pl.ds(base + full * CH, rem)])

    @pl.when(s == NS - 1)
    def _zero_tail():
        pltpu.sync_copy(rows_v.at[pl.ds(0, TAIL_ROWS)],
                        acc.at[pl.ds(TAIL_BASE, TAIL_ROWS)])

    plsc.subcore_barrier()

    def _chunk(k, carry):
        @pl.when(k == SHALF)
        def _restage():
            pltpu.sync_copy(srcp_hbm.at[wid].at[pl.ds(SHALF, SHALF)], src_v)
            pltpu.sync_copy(dstp_hbm.at[wid].at[pl.ds(SHALF, SHALF)], dst_v)
            pltpu.sync_copy(ewp_hbm.at[wid].at[pl.ds(SHALF, SHALF)], ew_v)

        kk = lax.rem(k, SHALF)
        pltpu.async_copy(table_hbm.at[src_v.at[kk]], rows_v, sem).wait()

        def _group(g, c2):
            wvec = ew_v[kk, pl.ds(g * 16, 16)]
            for l in range(16):
                w = wvec[l]
                for j in range(8):
                    sl = pl.ds(j * 16, 16)
                    rows_v[g * 16 + l, sl] = rows_v[g * 16 + l, sl] * w
            return c2

        lax.fori_loop(0, CH // 16, _group, 0)
        pltpu.sync_copy(rows_v, acc.at[dst_v.at[kk]], add=True)
        return carry

    nk = jnp.where(c == 0, KCH0, KCH1)
    lax.fori_loop(0, nk, _chunk, 0)
    plsc.subcore_barrier()
    pltpu.sync_copy(acc.at[pl.ds(base, ROWS_PER_TILE)],
                    out_hbm.at[c].at[pl.ds(base, ROWS_PER_TILE)])

    @pl.when(s == NS - 1)
    def _write_tail():
        pltpu.sync_copy(acc.at[pl.ds(TAIL_BASE, TAIL_ROWS)],
                        out_hbm.at[c].at[pl.ds(TAIL_BASE, TAIL_ROWS)])


def _post_body(p_ref, degc_ref, linw_ref, linb_ref, out_ref):
    dinv = lax.rsqrt(jnp.maximum(degc_ref[...], 1e-6))
    z = (p_ref[0] + p_ref[1]) * dinv
    h = jnp.maximum(z, 0.0)
    out_ref[...] = (jnp.dot(h, linw_ref[...],
                            preferred_element_type=jnp.float32)
                    + linb_ref[...])


_post = pl.pallas_call(
    _post_body,
    out_shape=jax.ShapeDtypeStruct((N, 1), jnp.float32),
)


def _worker_layout():
    counts = np.array([(KCH0 if (w % NC) == 0 else KCH1) * CH
                       for w in range(NW)], dtype=np.int64)
    starts = np.concatenate([[0], np.cumsum(counts)[:-1]])
    pos = starts[:, None] + np.arange(KPAD * CH)[None, :]
    valid = (np.arange(KPAD * CH)[None, :] < counts[:, None]) & (pos < E)
    pos = np.where(valid, pos, 0).astype(np.int32)
    return pos, valid.astype(np.float32)


_POS, _VALID = _worker_layout()


def kernel(x, edge, edge_weight, prev_hidden_state, deg, gcn_weight, lstm_c,
           W_xi, W_hi, b_i, W_xf, W_hf, b_f, W_xg, W_hg, b_g,
           W_xo, W_ho, b_o, lin_w, lin_b):
    src = edge[0, 0]
    dst = edge[0, 1]
    ew = edge_weight[0]
    degc = deg[1].reshape(N, 1)

    srcp = src[_POS].reshape(NW, KPAD, CH)
    dstp = dst[_POS].reshape(NW, KPAD, CH)
    ewp = (ew[_POS] * _VALID).reshape(NW, KPAD, CH)

    table = _prep(x, degc, gcn_weight, lstm_c,
                  W_xi, W_hi, b_i.reshape(1, F),
                  W_xf, W_hf, b_f.reshape(1, F),
                  W_xg, W_hg, b_g.reshape(1, F),
                  W_xo, W_ho, b_o.reshape(1, F))
    parts = _make_sc_scatter()(table, srcp, dstp, ewp)
    return _post(parts, degc, lin_w, lin_b.reshape(1, 1))


# speedup vs baseline: 5.9081x; 5.9081x over previous
import functools

import numpy as np

import jax
import jax.numpy as jnp
from jax import lax
from jax.experimental import pallas as pl
from jax.experimental.pallas import tpu as pltpu
from jax.experimental.pallas import tpu_sc as plsc

N = 10000
F = 128
E = 320000
NC = 2
NS = 16
NW = NC * NS
CH = 128
NCHUNKS = -(-E // CH)
KCH0 = int(round(NCHUNKS * 0.38 / NS))
KCH1 = -(-(NCHUNKS - KCH0 * NS) // NS)
KMAX = max(KCH0, KCH1)
SHALF = (-(-KMAX // 2) + 7) // 8 * 8
KPAD = 2 * SHALF
ROWS_PER_TILE = (N // NS) // 8 * 8
TAIL_ROWS = N - NS * ROWS_PER_TILE
TAIL_BASE = NS * ROWS_PER_TILE


def _prep_body(x_ref, degc_ref, w_ref, c_ref,
               wxi_ref, whi_ref, bi_ref, wxf_ref, whf_ref, bf_ref,
               wxg_ref, whg_ref, bg_ref, wxo_ref, who_ref, bo_ref,
               table_ref):
    W = w_ref[...]
    dot = functools.partial(jnp.dot, preferred_element_type=jnp.float32,
                            precision=lax.Precision.HIGHEST)
    i_g = jax.nn.sigmoid(dot(W, wxi_ref[...] + whi_ref[...]) + bi_ref[...])
    f_g = jax.nn.sigmoid(dot(W, wxf_ref[...] + whf_ref[...]) + bf_ref[...])
    g_g = jnp.tanh(dot(W, wxg_ref[...] + whg_ref[...]) + bg_ref[...])
    o_g = jax.nn.sigmoid(dot(W, wxo_ref[...] + who_ref[...]) + bo_ref[...])
    c_new = f_g * c_ref[...] + i_g * g_g
    W_new = o_g * jnp.tanh(c_new)
    dinv = lax.rsqrt(jnp.maximum(degc_ref[...], 1e-6))
    table_ref[...] = dot(x_ref[...] * dinv, W_new)


_prep = pl.pallas_call(
    _prep_body,
    out_shape=jax.ShapeDtypeStruct((N, F), jnp.float32),
)


@functools.cache
def _make_sc_scatter():
    mesh = plsc.VectorSubcoreMesh(core_axis_name="c", subcore_axis_name="s")
    return pl.kernel(
        _sc_scatter_body,
        out_type=jax.ShapeDtypeStruct((NC, N, F), jnp.float32),
        mesh=mesh,
        scratch_types=[
            pltpu.VMEM((SHALF, CH), jnp.int32),
            pltpu.VMEM((SHALF, CH), jnp.int32),
            pltpu.VMEM((SHALF, CH), jnp.float32),
            pltpu.VMEM((CH, F), jnp.float32),
            pltpu.VMEM_SHARED((N, F), jnp.float32),
            pltpu.SemaphoreType.DMA,
        ],
    )


def _sc_scatter_body(table_hbm, srcp_hbm, dstp_hbm, ewp_hbm, out_hbm,
                     src_v, dst_v, ew_v, rows_v, acc, sem):
    c = lax.axis_index("c")
    s = lax.axis_index("s")
    wid = s * NC + c
    pltpu.sync_copy(srcp_hbm.at[wid].at[pl.ds(0, SHALF)], src_v)
    pltpu.sync_copy(dstp_hbm.at[wid].at[pl.ds(0, SHALF)], dst_v)
    pltpu.sync_copy(ewp_hbm.at[wid].at[pl.ds(0, SHALF)], ew_v)

    zvec = jnp.zeros((16,), jnp.float32)

    def _zrow(i, carry):
        for j in range(8):
            rows_v[i, pl.ds(j * 16, 16)] = zvec
        return carry

    lax.fori_loop(0, CH, _zrow, 0)
    base = s * ROWS_PER_TILE
    full, rem = divmod(ROWS_PER_TILE, CH)
    for t in range(full):
        pltpu.sync_copy(rows_v, acc.at[pl.ds(base + t * CH, CH)])
    if rem:
        pltpu.sync_copy(rows_v.at[pl.ds(0, rem)],
                        acc.at[pl.ds(base + full * CH, rem)])

    @pl.when(s == NS - 1)
    def _zero_tail():
        pltpu.sync_copy(rows_v.at[pl.ds(0, TAIL_ROWS)],
                        acc.at[pl.ds(TAIL_BASE, TAIL_ROWS)])

    plsc.subcore_barrier()

    def _chunk(k, carry):
        @pl.when(k == SHALF)
        def _restage():
            pltpu.sync_copy(srcp_hbm.at[wid].at[pl.ds(SHALF, SHALF)], src_v)
            pltpu.sync_copy(dstp_hbm.at[wid].at[pl.ds(SHALF, SHALF)], dst_v)
            pltpu.sync_copy(ewp_hbm.at[wid].at[pl.ds(SHALF, SHALF)], ew_v)

        kk = lax.rem(k, SHALF)
        pltpu.async_copy(table_hbm.at[src_v.at[kk]], rows_v, sem).wait()

        def _group(g, c2):
            wvec = ew_v[kk, pl.ds(g * 16, 16)]
            for l in range(16):
                w = wvec[l]
                for j in range(8):
                    sl = pl.ds(j * 16, 16)
                    rows_v[g * 16 + l, sl] = rows_v[g * 16 + l, sl] * w
            return c2

        lax.fori_loop(0, CH // 16, _group, 0)
        pltpu.sync_copy(rows_v, acc.at[dst_v.at[kk]], add=True)
        return carry

    nk = jnp.where(c == 0, KCH0, KCH1)
    lax.fori_loop(0, nk, _chunk, 0)
    plsc.subcore_barrier()
    pltpu.sync_copy(acc.at[pl.ds(base, ROWS_PER_TILE)],
                    out_hbm.at[c].at[pl.ds(base, ROWS_PER_TILE)])

    @pl.when(s == NS - 1)
    def _write_tail():
        pltpu.sync_copy(acc.at[pl.ds(TAIL_BASE, TAIL_ROWS)],
                        out_hbm.at[c].at[pl.ds(TAIL_BASE, TAIL_ROWS)])


def _post_body(p_ref, degc_ref, linw_ref, linb_ref, out_ref):
    dinv = lax.rsqrt(jnp.maximum(degc_ref[...], 1e-6))
    z = (p_ref[0] + p_ref[1]) * dinv
    h = jnp.maximum(z, 0.0)
    out_ref[...] = (jnp.dot(h, linw_ref[...],
                            preferred_element_type=jnp.float32)
                    + linb_ref[...])


_post = pl.pallas_call(
    _post_body,
    out_shape=jax.ShapeDtypeStruct((N, 1), jnp.float32),
)


E0 = NS * KCH0 * CH
E1 = NS * KCH1 * CH


def _layout(a, fill_dtype):
    a0 = a[:E0].reshape(NS, KCH0, CH)
    a0 = jnp.pad(a0, ((0, 0), (0, KPAD - KCH0), (0, 0)))
    a1 = jnp.pad(a[E0:], (0, E0 + E1 - a.shape[0])).reshape(NS, KCH1, CH)
    a1 = jnp.pad(a1, ((0, 0), (0, KPAD - KCH1), (0, 0)))
    return jnp.stack([a0, a1], axis=1).reshape(NW, KPAD, CH)


def kernel(x, edge, edge_weight, prev_hidden_state, deg, gcn_weight, lstm_c,
           W_xi, W_hi, b_i, W_xf, W_hf, b_f, W_xg, W_hg, b_g,
           W_xo, W_ho, b_o, lin_w, lin_b):
    src = edge[0, 0]
    dst = edge[0, 1]
    ew = edge_weight[0]
    degc = deg[1].reshape(N, 1)

    srcp = _layout(src, jnp.int32)
    dstp = _layout(dst, jnp.int32)
    ewp = _layout(ew, jnp.float32)

    table = _prep(x, degc, gcn_weight, lstm_c,
                  W_xi, W_hi, b_i.reshape(1, F),
                  W_xf, W_hf, b_f.reshape(1, F),
                  W_xg, W_hg, b_g.reshape(1, F),
                  W_xo, W_ho, b_o.reshape(1, F))
    parts = _make_sc_scatter()(table, srcp, dstp, ewp)
    return _post(parts, degc, lin_w, lin_b.reshape(1, 1))
